# baseline (speedup 1.0000x reference)
import functools

import jax
import jax.numpy as jnp
from jax import lax
from jax.experimental import pallas as pl
from jax.experimental.pallas import tpu as pltpu
from jax.experimental.pallas import tpu_sc as plsc

N = 10000
D_HID = 64
DEG_W = 16
NC = 2
NS = 16
NW = NC * NS
CHUNK = 128
NBUF = 5
Q0 = 160
Q1 = 0
IDX_BLK = 80
N_PAD = 10240
RPS = N_PAD // NS
EPS = 1e-5


def _sc_mesh():
    return plsc.VectorSubcoreMesh(core_axis_name="c", subcore_axis_name="s")


def _zero_rows(ref, n_rows, width):
    zero16 = jnp.zeros((16,), jnp.float32)

    def body(i, _):
        for j in range(width // 16):
            ref[i, pl.ds(j * 16, 16)] = zero16
        return 0

    lax.fori_loop(0, n_rows, body, 0)


def _sc_aggregate(p, src, dst, q0, q1, with_deg):
    assert q0 % IDX_BLK == 0 and q1 % IDX_BLK == 0
    assert IDX_BLK % NBUF == 0
    assert NBUF * CHUNK == RPS

    out_type = [jax.ShapeDtypeStruct((2 * N_PAD, D_HID), jnp.float32)]
    scratch = [
        pltpu.VMEM((IDX_BLK, CHUNK), jnp.int32),
        pltpu.VMEM((IDX_BLK, CHUNK), jnp.int32),
        pltpu.VMEM((NBUF * CHUNK, D_HID), jnp.float32),
    ]
    scratch += [pltpu.SemaphoreType.DMA for _ in range(NBUF)]
    scratch.append(pltpu.VMEM_SHARED((N_PAD, D_HID), jnp.float32))
    if with_deg:
        out_type.append(jax.ShapeDtypeStruct((2 * N_PAD, DEG_W), jnp.float32))
        scratch += [
            pltpu.VMEM((CHUNK, DEG_W), jnp.float32),
            pltpu.VMEM_SHARED((N_PAD, DEG_W), jnp.float32),
        ]

    def body(p_hbm, src_hbm, dst_hbm, *rest):
        if with_deg:
            (acc_out, deg_out, src_v, dst_v, rows_v, *tail) = rest
            sems = tail[:NBUF]
            acc_sh, ones_v, deg_sh = tail[NBUF:]
        else:
            (acc_out, src_v, dst_v, rows_v, *tail) = rest
            sems = tail[:NBUF]
            (acc_sh,) = tail[NBUF:]

        cid = lax.axis_index("c")
        sid = lax.axis_index("s")
        row0 = sid * RPS
        quota = jnp.where(cid == 0, q0, q1)
        tile_base = jnp.where(cid == 0, sid * q0, NS * q0 + sid * q1)

        _zero_rows(rows_v, RPS, D_HID)
        pltpu.sync_copy(rows_v, acc_sh.at[pl.ds(row0, RPS)])
        if with_deg:
            _zero_rows(ones_v, CHUNK, DEG_W)
            for k in range(RPS // CHUNK):
                pltpu.sync_copy(ones_v, deg_sh.at[pl.ds(row0 + k * CHUNK, CHUNK)])
            one16 = jnp.ones((16,), jnp.float32)

            def fill_ones(i, _):
                ones_v[i, :] = one16
                return 0

            lax.fori_loop(0, CHUNK, fill_ones, 0)
        plsc.subcore_barrier()

        def idx_block(blk, _):
            off = tile_base + blk * IDX_BLK
            pltpu.sync_copy(src_hbm.at[pl.ds(off, IDX_BLK)], src_v)
            pltpu.sync_copy(dst_hbm.at[pl.ds(off, IDX_BLK)], dst_v)

            def group(i, _):
                c0 = i * NBUF
                gathers = [
                    pltpu.async_copy(
                        p_hbm.at[src_v.at[c0 + b]],
                        rows_v.at[pl.ds(b * CHUNK, CHUNK)], sems[b])
                    for b in range(NBUF)
                ]
                for b in range(NBUF):
                    gathers[b].wait()
                    pltpu.sync_copy(rows_v.at[pl.ds(b * CHUNK, CHUNK)],
                                    acc_sh.at[dst_v.at[c0 + b]], add=True)
                    if with_deg:
                        pltpu.sync_copy(ones_v, deg_sh.at[dst_v.at[c0 + b]],
                                        add=True)
                return 0

            lax.fori_loop(0, IDX_BLK // NBUF, group, 0)
            return 0

        lax.fori_loop(0, quota // IDX_BLK, idx_block, 0)
        plsc.subcore_barrier()

        out0 = cid * N_PAD + row0
        pltpu.sync_copy(acc_sh.at[pl.ds(row0, RPS)], acc_out.at[pl.ds(out0, RPS)])
        if with_deg:
            pltpu.sync_copy(deg_sh.at[pl.ds(row0, RPS)], deg_out.at[pl.ds(out0, RPS)])

    fn = pl.kernel(
        body,
        out_type=tuple(out_type),
        mesh=_sc_mesh(),
        scratch_types=tuple(scratch),
        compiler_params=pltpu.CompilerParams(use_tc_tiling_on_sc=False),
    )
    res = fn(p, src, dst)
    if with_deg:
        return res
    return res[0] if isinstance(res, (tuple, list)) else res


def _project2(x, wl, wr):
    def body(x_ref, wl_ref, wr_ref, p_ref, r_ref):
        xv = x_ref[...]
        p_ref[...] = jnp.dot(xv, wl_ref[...], preferred_element_type=jnp.float32)
        r_ref[...] = jnp.dot(xv, wr_ref[...], preferred_element_type=jnp.float32)

    d = wl.shape[1]
    return pl.pallas_call(
        body,
        out_shape=[jax.ShapeDtypeStruct((N, d), jnp.float32)] * 2,
    )(x, wl, wr)


def _sum_parts(acc_ref, lo, hi):
    return acc_ref[0:N, lo:hi] + acc_ref[N_PAD:N_PAD + N, lo:hi]


def _deg_from_parts(deg_ref):
    d = _sum_parts(deg_ref, 0, DEG_W)
    return jnp.max(d, axis=1, keepdims=True)


def _bn_relu(pre, g, be):
    mu = jnp.mean(pre, axis=0, keepdims=True)
    var = jnp.mean((pre - mu) ** 2, axis=0, keepdims=True)
    h = g * (pre - mu) * lax.rsqrt(var + EPS) + be
    return jnp.maximum(h, 0.0)


def _layer1_post(acc1, deg, r, b1, g1, be1):
    def body(acc_ref, deg_ref, r_ref, b_ref, g_ref, be_ref, h_ref):
        a = _sum_parts(acc_ref, 0, D_HID)
        degv = _deg_from_parts(deg_ref)
        pre = a / jnp.maximum(degv, 1.0) + r_ref[...] + b_ref[...]
        h_ref[...] = _bn_relu(pre, g_ref[...], be_ref[...])

    return pl.pallas_call(
        body,
        out_shape=jax.ShapeDtypeStruct((N, D_HID), jnp.float32),
    )(acc1, deg, r, b1, g1, be1)


def _layer2_post(acc2, deg, h, wl, wr, b2, g2, be2):
    def body(acc2_ref, deg_ref, h_ref, wl_ref, wr_ref, b_ref, g_ref, be_ref, o_ref):
        a = _sum_parts(acc2_ref, 0, D_HID)
        degv = _deg_from_parts(deg_ref)
        agg = a / jnp.maximum(degv, 1.0)
        z = (jnp.dot(agg, wl_ref[...], preferred_element_type=jnp.float32)
             + jnp.dot(h_ref[...], wr_ref[...], preferred_element_type=jnp.float32)
             + b_ref[...])
        o_ref[...] = _bn_relu(z, g_ref[...], be_ref[...])

    d_out = wl.shape[1]
    return pl.pallas_call(
        body,
        out_shape=jax.ShapeDtypeStruct((N, d_out), jnp.float32),
    )(acc2, deg, h, wl, wr, b2, g2, be2)


def kernel(x, edge_index, W1_l, W1_r, b1, g1, be1, W2_l, W2_r, b2, g2, be2):
    ei = edge_index.astype(jnp.int32)
    n_edges = ei.shape[1]
    total_chunks = NS * (Q0 + Q1)
    assert total_chunks * CHUNK >= n_edges
    rows_total = total_chunks
    pad = rows_total * CHUNK - n_edges
    src = jnp.concatenate([ei[0], jnp.zeros((pad,), jnp.int32)])
    dst = jnp.concatenate([ei[1], jnp.full((pad,), N, jnp.int32)])
    src = src.reshape(rows_total, CHUNK)
    dst = dst.reshape(rows_total, CHUNK)

    b1r, g1r, be1r = b1.reshape(1, -1), g1.reshape(1, -1), be1.reshape(1, -1)
    b2r, g2r, be2r = b2.reshape(1, -1), g2.reshape(1, -1), be2.reshape(1, -1)

    p1, r1 = _project2(x, W1_l, W1_r)
    acc1, deg = _sc_aggregate(p1, src, dst, Q0, Q1, with_deg=True)
    h = _layer1_post(acc1, deg, r1, b1r, g1r, be1r)
    acc2 = _sc_aggregate(h, src, dst, Q0, Q1, with_deg=False)
    return _layer2_post(acc2, deg, h, W2_l, W2_r, b2r, g2r, be2r)


# speedup vs baseline: 1.2985x; 1.2985x over previous
import functools

import jax
import jax.numpy as jnp
from jax import lax
from jax.experimental import pallas as pl
from jax.experimental.pallas import tpu as pltpu
from jax.experimental.pallas import tpu_sc as plsc

N = 10000
D_HID = 64
DEG_W = 16
NC = 2
NS = 16
NW = NC * NS
CHUNK = 128
NBUF = 5
Q0 = 120
Q1 = 40
N_PAD = 10240
RPS = N_PAD // NS
EPS = 1e-5


def _sc_mesh():
    return plsc.VectorSubcoreMesh(core_axis_name="c", subcore_axis_name="s")


def _zero_rows(ref, n_rows, width):
    zero16 = jnp.zeros((16,), jnp.float32)

    def body(i, _):
        for j in range(width // 16):
            ref[i, pl.ds(j * 16, 16)] = zero16
        return 0

    lax.fori_loop(0, n_rows, body, 0)


def _sc_aggregate(p, src, dst, q0, q1, with_deg):
    assert q0 % NBUF == 0 and q1 % NBUF == 0
    assert NBUF * CHUNK == RPS
    qmax = max(q0, q1)

    out_type = [jax.ShapeDtypeStruct((2 * N_PAD, D_HID), jnp.float32)]
    scratch = [
        pltpu.VMEM((qmax, CHUNK), jnp.int32),
        pltpu.VMEM((qmax, CHUNK), jnp.int32),
        pltpu.VMEM((NBUF * CHUNK, D_HID), jnp.float32),
    ]
    scratch += [pltpu.SemaphoreType.DMA for _ in range(NBUF)]
    scratch.append(pltpu.VMEM_SHARED((N_PAD, D_HID), jnp.float32))
    if with_deg:
        out_type.append(jax.ShapeDtypeStruct((2 * N_PAD, DEG_W), jnp.float32))
        scratch += [
            pltpu.VMEM((CHUNK, DEG_W), jnp.float32),
            pltpu.VMEM_SHARED((N_PAD, DEG_W), jnp.float32),
        ]

    def body(p_hbm, src_hbm, dst_hbm, *rest):
        if with_deg:
            (acc_out, deg_out, src_v, dst_v, rows_v, *tail) = rest
            sems = tail[:NBUF]
            acc_sh, ones_v, deg_sh = tail[NBUF:]
        else:
            (acc_out, src_v, dst_v, rows_v, *tail) = rest
            sems = tail[:NBUF]
            (acc_sh,) = tail[NBUF:]

        cid = lax.axis_index("c")
        sid = lax.axis_index("s")
        row0 = sid * RPS
        quota = jnp.where(cid == 0, q0, q1)
        tile_base = jnp.where(cid == 0, sid * q0, NS * q0 + sid * q1)

        pltpu.sync_copy(src_hbm.at[pl.ds(tile_base, qmax)], src_v)
        pltpu.sync_copy(dst_hbm.at[pl.ds(tile_base, qmax)], dst_v)

        _zero_rows(rows_v, RPS, D_HID)
        pltpu.sync_copy(rows_v, acc_sh.at[pl.ds(row0, RPS)])
        if with_deg:
            _zero_rows(ones_v, CHUNK, DEG_W)
            for k in range(RPS // CHUNK):
                pltpu.sync_copy(ones_v, deg_sh.at[pl.ds(row0 + k * CHUNK, CHUNK)])
            one16 = jnp.ones((16,), jnp.float32)

            def fill_ones(i, _):
                ones_v[i, :] = one16
                return 0

            lax.fori_loop(0, CHUNK, fill_ones, 0)
        plsc.subcore_barrier()

        def group(i, _):
            c0 = i * NBUF
            gathers = [
                pltpu.async_copy(
                    p_hbm.at[src_v.at[c0 + b]],
                    rows_v.at[pl.ds(b * CHUNK, CHUNK)], sems[b])
                for b in range(NBUF)
            ]
            for b in range(NBUF):
                gathers[b].wait()
                pltpu.sync_copy(rows_v.at[pl.ds(b * CHUNK, CHUNK)],
                                acc_sh.at[dst_v.at[c0 + b]], add=True)
                if with_deg:
                    pltpu.sync_copy(ones_v, deg_sh.at[dst_v.at[c0 + b]],
                                    add=True)
            return 0

        lax.fori_loop(0, quota // NBUF, group, 0)
        plsc.subcore_barrier()

        out0 = cid * N_PAD + row0
        pltpu.sync_copy(acc_sh.at[pl.ds(row0, RPS)], acc_out.at[pl.ds(out0, RPS)])
        if with_deg:
            pltpu.sync_copy(deg_sh.at[pl.ds(row0, RPS)], deg_out.at[pl.ds(out0, RPS)])

    fn = pl.kernel(
        body,
        out_type=tuple(out_type),
        mesh=_sc_mesh(),
        scratch_types=tuple(scratch),
        compiler_params=pltpu.CompilerParams(use_tc_tiling_on_sc=False),
    )
    res = fn(p, src, dst)
    if with_deg:
        return res
    return res[0] if isinstance(res, (tuple, list)) else res


def _project2(x, wl, wr):
    def body(x_ref, wl_ref, wr_ref, p_ref, r_ref):
        xv = x_ref[...]
        p_ref[...] = jnp.dot(xv, wl_ref[...], preferred_element_type=jnp.float32)
        r_ref[...] = jnp.dot(xv, wr_ref[...], preferred_element_type=jnp.float32)

    d = wl.shape[1]
    return pl.pallas_call(
        body,
        out_shape=[jax.ShapeDtypeStruct((N, d), jnp.float32)] * 2,
    )(x, wl, wr)


def _sum_parts(acc_ref, lo, hi):
    return acc_ref[0:N, lo:hi] + acc_ref[N_PAD:N_PAD + N, lo:hi]


def _deg_from_parts(deg_ref):
    d = _sum_parts(deg_ref, 0, DEG_W)
    return jnp.max(d, axis=1, keepdims=True)


def _bn_relu(pre, g, be):
    mu = jnp.mean(pre, axis=0, keepdims=True)
    var = jnp.mean((pre - mu) ** 2, axis=0, keepdims=True)
    h = g * (pre - mu) * lax.rsqrt(var + EPS) + be
    return jnp.maximum(h, 0.0)


def _layer1_post(acc1, deg, r, b1, g1, be1):
    def body(acc_ref, deg_ref, r_ref, b_ref, g_ref, be_ref, h_ref):
        a = _sum_parts(acc_ref, 0, D_HID)
        degv = _deg_from_parts(deg_ref)
        pre = a / jnp.maximum(degv, 1.0) + r_ref[...] + b_ref[...]
        h_ref[...] = _bn_relu(pre, g_ref[...], be_ref[...])

    return pl.pallas_call(
        body,
        out_shape=jax.ShapeDtypeStruct((N, D_HID), jnp.float32),
    )(acc1, deg, r, b1, g1, be1)


def _layer2_post(acc2, deg, h, wl, wr, b2, g2, be2):
    def body(acc2_ref, deg_ref, h_ref, wl_ref, wr_ref, b_ref, g_ref, be_ref, o_ref):
        a = _sum_parts(acc2_ref, 0, D_HID)
        degv = _deg_from_parts(deg_ref)
        agg = a / jnp.maximum(degv, 1.0)
        z = (jnp.dot(agg, wl_ref[...], preferred_element_type=jnp.float32)
             + jnp.dot(h_ref[...], wr_ref[...], preferred_element_type=jnp.float32)
             + b_ref[...])
        o_ref[...] = _bn_relu(z, g_ref[...], be_ref[...])

    d_out = wl.shape[1]
    return pl.pallas_call(
        body,
        out_shape=jax.ShapeDtypeStruct((N, d_out), jnp.float32),
    )(acc2, deg, h, wl, wr, b2, g2, be2)


def kernel(x, edge_index, W1_l, W1_r, b1, g1, be1, W2_l, W2_r, b2, g2, be2):
    ei = edge_index.astype(jnp.int32)
    n_edges = ei.shape[1]
    total_chunks = NS * (Q0 + Q1)
    assert total_chunks * CHUNK >= n_edges
    rows_total = total_chunks + max(Q0, Q1)
    pad = rows_total * CHUNK - n_edges
    src = jnp.concatenate([ei[0], jnp.zeros((pad,), jnp.int32)])
    dst = jnp.concatenate([ei[1], jnp.full((pad,), N, jnp.int32)])
    src = src.reshape(rows_total, CHUNK)
    dst = dst.reshape(rows_total, CHUNK)

    b1r, g1r, be1r = b1.reshape(1, -1), g1.reshape(1, -1), be1.reshape(1, -1)
    b2r, g2r, be2r = b2.reshape(1, -1), g2.reshape(1, -1), be2.reshape(1, -1)

    p1, r1 = _project2(x, W1_l, W1_r)
    acc1, deg = _sc_aggregate(p1, src, dst, Q0, Q1, with_deg=True)
    h = _layer1_post(acc1, deg, r1, b1r, g1r, be1r)
    acc2 = _sc_aggregate(h, src, dst, Q0, Q1, with_deg=False)
    return _layer2_post(acc2, deg, h, W2_l, W2_r, b2r, g2r, be2r)
